# baseline (speedup 1.0000x reference)
import functools

import jax
import jax.numpy as jnp
from jax import lax
from jax.experimental import pallas as pl
from jax.experimental.pallas import tpu as pltpu
from jax.experimental.pallas import tpu_sc as plsc

B = 16384
D = 64
O = 16
N = 1000000
_PBLK = 65536
_NBLK = (N + _PBLK - 1) // _PBLK
_YROWS = _PBLK // 8
G = _NBLK * _YROWS
_SH_B = _PBLK.bit_length() - 1
_SH_K = _YROWS.bit_length() - 1
_MASK_G = _YROWS - 1

_info = plsc.get_sparse_core_info()
_NC, _NS = _info.num_cores, _info.num_subcores
_NW = _NC * _NS
_BPW = B // _NW
_L = 16


def _proj_body(tT_ref, w_ref, b_ref, y_ref):
    yt = jnp.dot(w_ref[...], tT_ref[...], preferred_element_type=jnp.float32)
    row = lax.broadcasted_iota(jnp.int32, (O, 128), 0)
    col = lax.broadcasted_iota(jnp.int32, (O, 128), 1)
    y = b_ref[...]
    for k in range(8):
        ek = (col == row + 16 * k).astype(jnp.float32)
        y = y + jax.lax.dot_general(
            yt[:, k * _YROWS : (k + 1) * _YROWS],
            ek,
            dimension_numbers=(((0,), (0,)), ((), ())),
            preferred_element_type=jnp.float32,
        )
    y_ref[...] = y


def _project(tT, w, b128):
    return pl.pallas_call(
        _proj_body,
        grid=(_NBLK,),
        in_specs=[
            pl.BlockSpec((D, _PBLK), lambda i: (0, i)),
            pl.BlockSpec((O, D), lambda i: (0, 0)),
            pl.BlockSpec((1, 128), lambda i: (0, 0)),
        ],
        out_specs=pl.BlockSpec((_YROWS, 128), lambda i: (i, 0)),
        out_shape=jax.ShapeDtypeStruct((G, 128), jnp.float32),
        compiler_params=pltpu.CompilerParams(fuse_transposed_lhs_in_matmul=True),
    )(tT, w, b128)


def _make_gather():
    mesh = plsc.VectorSubcoreMesh(core_axis_name="c", subcore_axis_name="s")

    @functools.partial(
        pl.kernel,
        mesh=mesh,
        out_type=jax.ShapeDtypeStruct((O, B), jnp.float32),
        scratch_types=[
            pltpu.VMEM((_BPW,), jnp.int32),
            pltpu.VMEM((_BPW,), jnp.int32),
            pltpu.VMEM((_BPW, 128), jnp.float32),
            pltpu.VMEM((O, _BPW), jnp.float32),
            pltpu.SemaphoreType.DMA,
        ],
        compiler_params=pltpu.CompilerParams(needs_layout_passes=False),
    )
    def gather_k(y_hbm, ids_hbm, outT_hbm, ids_v, idx_v, rows_v, sel_v, sem):
        wid = lax.axis_index("s") * _NC + lax.axis_index("c")
        base = wid * _BPW
        pltpu.sync_copy(ids_hbm.at[pl.ds(base, _BPW)], ids_v)

        def mk_idx(j, _):
            n = ids_v[pl.ds(j * _L, _L)]
            idx_v[pl.ds(j * _L, _L)] = ((n >> _SH_B) << _SH_K) | (n & _MASK_G)
            return _

        lax.fori_loop(0, _BPW // _L, mk_idx, None)
        pltpu.async_copy(y_hbm.at[idx_v], rows_v, sem).wait()

        lane = lax.iota(jnp.int32, _L)

        def select(j, _):
            n = ids_v[pl.ds(j * _L, _L)]
            sub = ((n >> _SH_K) & 7) * 16
            row_idx = j * _L + lane
            for o in range(O):
                vals = plsc.load_gather(rows_v, [row_idx, sub + o])
                plsc.store_scatter(sel_v, [lane * 0 + o, row_idx], vals)
            return _

        lax.fori_loop(0, _BPW // _L, select, None)
        pltpu.sync_copy(sel_v, outT_hbm.at[:, pl.ds(base, _BPW)])

    return gather_k


_gather = _make_gather()


def kernel(ids, table, W, b):
    ids = ids.astype(jnp.int32)
    b128 = jnp.tile(b, 8).reshape(1, 128)
    y = _project(table.T, W, b128)
    outT = _gather(y, ids)
    return outT.T


# speedup vs baseline: 1.0383x; 1.0383x over previous
import functools

import jax
import jax.numpy as jnp
from jax import lax
from jax.experimental import pallas as pl
from jax.experimental.pallas import tpu as pltpu
from jax.experimental.pallas import tpu_sc as plsc

B = 16384
D = 64
O = 16
N = 1000000
_PBLK = 32768
_NBLK = (N + _PBLK - 1) // _PBLK
_YROWS = _PBLK // 8
G = _NBLK * _YROWS
_SH_B = _PBLK.bit_length() - 1
_SH_K = _YROWS.bit_length() - 1
_MASK_G = _YROWS - 1

_info = plsc.get_sparse_core_info()
_NC, _NS = _info.num_cores, _info.num_subcores
_NW = _NC * _NS
_BPW = B // _NW
_L = 16


def _proj_body(tT_ref, w_ref, b_ref, y_ref):
    yt = jnp.dot(w_ref[...], tT_ref[...], preferred_element_type=jnp.float32)
    row = lax.broadcasted_iota(jnp.int32, (O, 128), 0)
    col = lax.broadcasted_iota(jnp.int32, (O, 128), 1)
    y = b_ref[...]
    for k in range(8):
        ek = (col == row + 16 * k).astype(jnp.float32)
        y = y + jax.lax.dot_general(
            yt[:, k * _YROWS : (k + 1) * _YROWS],
            ek,
            dimension_numbers=(((0,), (0,)), ((), ())),
            preferred_element_type=jnp.float32,
        )
    y_ref[...] = y


def _project(tT, w, b128):
    return pl.pallas_call(
        _proj_body,
        grid=(_NBLK,),
        in_specs=[
            pl.BlockSpec((D, _PBLK), lambda i: (0, i)),
            pl.BlockSpec((O, D), lambda i: (0, 0)),
            pl.BlockSpec((1, 128), lambda i: (0, 0)),
        ],
        out_specs=pl.BlockSpec((_YROWS, 128), lambda i: (i, 0)),
        out_shape=jax.ShapeDtypeStruct((G, 128), jnp.float32),
        compiler_params=pltpu.CompilerParams(
            dimension_semantics=("parallel",),
            fuse_transposed_lhs_in_matmul=True,
        ),
    )(tT, w, b128)


def _make_gather():
    mesh = plsc.VectorSubcoreMesh(core_axis_name="c", subcore_axis_name="s")

    @functools.partial(
        pl.kernel,
        mesh=mesh,
        out_type=jax.ShapeDtypeStruct((O, B), jnp.float32),
        scratch_types=[
            pltpu.VMEM((_BPW,), jnp.int32),
            pltpu.VMEM((_BPW,), jnp.int32),
            pltpu.VMEM((_BPW, 128), jnp.float32),
            pltpu.VMEM((O, _BPW), jnp.float32),
            pltpu.SemaphoreType.DMA,
        ],
        compiler_params=pltpu.CompilerParams(needs_layout_passes=False),
    )
    def gather_k(y_hbm, ids_hbm, outT_hbm, ids_v, idx_v, rows_v, sel_v, sem):
        wid = lax.axis_index("s") * _NC + lax.axis_index("c")
        base = wid * _BPW
        pltpu.sync_copy(ids_hbm.at[pl.ds(base, _BPW)], ids_v)

        def mk_idx(j, _):
            n = ids_v[pl.ds(j * _L, _L)]
            idx_v[pl.ds(j * _L, _L)] = ((n >> _SH_B) << _SH_K) | (n & _MASK_G)
            return _

        lax.fori_loop(0, _BPW // _L, mk_idx, None)
        pltpu.async_copy(y_hbm.at[idx_v], rows_v, sem).wait()

        lane = lax.iota(jnp.int32, _L)

        def select(j, _):
            n = ids_v[pl.ds(j * _L, _L)]
            sub = ((n >> _SH_K) & 7) * 16
            row_idx = j * _L + lane
            for o in range(O):
                vals = plsc.load_gather(rows_v, [row_idx, sub + o])
                plsc.store_scatter(sel_v, [lane * 0 + o, row_idx], vals)
            return _

        lax.fori_loop(0, _BPW // _L, select, None)
        pltpu.sync_copy(sel_v, outT_hbm.at[:, pl.ds(base, _BPW)])

    return gather_k


_gather = _make_gather()


def kernel(ids, table, W, b):
    ids = ids.astype(jnp.int32)
    b128 = jnp.tile(b, 8).reshape(1, 128)
    y = _project(table.T, W, b128)
    outT = _gather(y, ids)
    return outT.T


# speedup vs baseline: 1.2630x; 1.2163x over previous
import functools

import jax
import jax.numpy as jnp
from jax import lax
from jax.experimental import pallas as pl
from jax.experimental.pallas import tpu as pltpu
from jax.experimental.pallas import tpu_sc as plsc

B = 16384
D = 64
O = 16
N = 1000000
_PBLK = 32768
_NBLK = (N + _PBLK - 1) // _PBLK
_YROWS = _PBLK // 8
G = _NBLK * _YROWS
_SH_B = _PBLK.bit_length() - 1
_SH_K = _YROWS.bit_length() - 1
_MASK_G = _YROWS - 1

_info = plsc.get_sparse_core_info()
_NC, _NS = _info.num_cores, _info.num_subcores
_NW = _NC * _NS
_BPW = B // _NW
_L = 16


def _proj_body(tT_ref, w_ref, b_ref, y_ref):
    yt = jnp.dot(w_ref[...], tT_ref[...], preferred_element_type=jnp.float32)
    row = lax.broadcasted_iota(jnp.int32, (O, 128), 0)
    col = lax.broadcasted_iota(jnp.int32, (O, 128), 1)
    yt_bf = yt.astype(jnp.bfloat16)
    y = b_ref[...]
    for k in range(8):
        ek = (col == row + 16 * k).astype(jnp.bfloat16)
        y = y + jax.lax.dot_general(
            yt_bf[:, k * _YROWS : (k + 1) * _YROWS],
            ek,
            dimension_numbers=(((0,), (0,)), ((), ())),
            preferred_element_type=jnp.float32,
        )
    y_ref[...] = y


def _project(tT, w, b128):
    return pl.pallas_call(
        _proj_body,
        grid=(_NBLK,),
        in_specs=[
            pl.BlockSpec((D, _PBLK), lambda i: (0, i)),
            pl.BlockSpec((O, D), lambda i: (0, 0)),
            pl.BlockSpec((1, 128), lambda i: (0, 0)),
        ],
        out_specs=pl.BlockSpec((_YROWS, 128), lambda i: (i, 0)),
        out_shape=jax.ShapeDtypeStruct((G, 128), jnp.float32),
        compiler_params=pltpu.CompilerParams(
            dimension_semantics=("parallel",),
            fuse_transposed_lhs_in_matmul=True,
        ),
    )(tT, w, b128)


def _make_gather():
    mesh = plsc.VectorSubcoreMesh(core_axis_name="c", subcore_axis_name="s")

    @functools.partial(
        pl.kernel,
        mesh=mesh,
        out_type=jax.ShapeDtypeStruct((O, B), jnp.float32),
        scratch_types=[
            pltpu.VMEM((_BPW,), jnp.int32),
            pltpu.VMEM((_BPW,), jnp.int32),
            pltpu.VMEM((_BPW, 128), jnp.float32),
            pltpu.VMEM((O, _BPW), jnp.float32),
            pltpu.SemaphoreType.DMA,
        ],
        compiler_params=pltpu.CompilerParams(needs_layout_passes=False),
    )
    def gather_k(y_hbm, ids_hbm, outT_hbm, ids_v, idx_v, rows_v, sel_v, sem):
        wid = lax.axis_index("s") * _NC + lax.axis_index("c")
        base = wid * _BPW
        pltpu.sync_copy(ids_hbm.at[pl.ds(base, _BPW)], ids_v)

        def mk_idx(j, _):
            n = ids_v[pl.ds(j * _L, _L)]
            idx_v[pl.ds(j * _L, _L)] = ((n >> _SH_B) << _SH_K) | (n & _MASK_G)
            return _

        lax.fori_loop(0, _BPW // _L, mk_idx, None)
        pltpu.async_copy(y_hbm.at[idx_v], rows_v, sem).wait()

        lane = lax.iota(jnp.int32, _L)

        def select(j, _):
            n = ids_v[pl.ds(j * _L, _L)]
            sub = ((n >> _SH_K) & 7) * 16
            row_idx = j * _L + lane
            for o in range(O):
                vals = plsc.load_gather(rows_v, [row_idx, sub + o])
                plsc.store_scatter(sel_v, [lane * 0 + o, row_idx], vals)
            return _

        lax.fori_loop(0, _BPW // _L, select, None)
        pltpu.sync_copy(sel_v, outT_hbm.at[:, pl.ds(base, _BPW)])

    return gather_k


_gather = _make_gather()


def kernel(ids, table, W, b):
    ids = ids.astype(jnp.int32)
    b128 = jnp.tile(b, 8).reshape(1, 128)
    y = _project(table.T, W, b128)
    outT = _gather(y, ids)
    return outT.T


# speedup vs baseline: 1.4536x; 1.1509x over previous
import functools

import jax
import jax.numpy as jnp
from jax import lax
from jax.experimental import pallas as pl
from jax.experimental.pallas import tpu as pltpu
from jax.experimental.pallas import tpu_sc as plsc

B = 16384
D = 64
O = 16
N = 1000000
_PBLK = 32768
_NBLK = (N + _PBLK - 1) // _PBLK
_YROWS = _PBLK // 8
G = _NBLK * _YROWS
_SH_B = _PBLK.bit_length() - 1
_SH_K = _YROWS.bit_length() - 1
_MASK_G = _YROWS - 1

_info = plsc.get_sparse_core_info()
_NC, _NS = _info.num_cores, _info.num_subcores
_NW = _NC * _NS
_BPW = B // _NW
_L = 16


def _proj_body(tT_ref, w_ref, b_ref, y_ref):
    yt = jnp.dot(w_ref[...], tT_ref[...], preferred_element_type=jnp.float32)
    yt_bf = yt.astype(jnp.bfloat16)
    l = jnp.concatenate(
        [yt_bf[:, k * _YROWS : (k + 1) * _YROWS] for k in range(8)], axis=0
    )
    row = lax.broadcasted_iota(jnp.int32, (128, 128), 0)
    col = lax.broadcasted_iota(jnp.int32, (128, 128), 1)
    i128 = (row == col).astype(jnp.bfloat16)
    y_ref[...] = b_ref[...] + jax.lax.dot_general(
        l,
        i128,
        dimension_numbers=(((0,), (0,)), ((), ())),
        preferred_element_type=jnp.float32,
    )


def _project(tT, w, b128):
    return pl.pallas_call(
        _proj_body,
        grid=(_NBLK,),
        in_specs=[
            pl.BlockSpec((D, _PBLK), lambda i: (0, i)),
            pl.BlockSpec((O, D), lambda i: (0, 0)),
            pl.BlockSpec((1, 128), lambda i: (0, 0)),
        ],
        out_specs=pl.BlockSpec((_YROWS, 128), lambda i: (i, 0)),
        out_shape=jax.ShapeDtypeStruct((G, 128), jnp.float32),
        compiler_params=pltpu.CompilerParams(
            dimension_semantics=("parallel",),
            fuse_transposed_lhs_in_matmul=True,
        ),
    )(tT, w, b128)


def _make_gather():
    mesh = plsc.VectorSubcoreMesh(core_axis_name="c", subcore_axis_name="s")

    @functools.partial(
        pl.kernel,
        mesh=mesh,
        out_type=jax.ShapeDtypeStruct((O, B), jnp.float32),
        scratch_types=[
            pltpu.VMEM((_BPW,), jnp.int32),
            pltpu.VMEM((_BPW,), jnp.int32),
            pltpu.VMEM((_BPW, 128), jnp.float32),
            pltpu.VMEM((O, _BPW), jnp.float32),
            pltpu.SemaphoreType.DMA,
        ],
        compiler_params=pltpu.CompilerParams(needs_layout_passes=False),
    )
    def gather_k(y_hbm, ids_hbm, outT_hbm, ids_v, idx_v, rows_v, sel_v, sem):
        wid = lax.axis_index("s") * _NC + lax.axis_index("c")
        base = wid * _BPW
        pltpu.sync_copy(ids_hbm.at[pl.ds(base, _BPW)], ids_v)

        def mk_idx(j, _):
            n = ids_v[pl.ds(j * _L, _L)]
            idx_v[pl.ds(j * _L, _L)] = ((n >> _SH_B) << _SH_K) | (n & _MASK_G)
            return _

        lax.fori_loop(0, _BPW // _L, mk_idx, None)
        pltpu.async_copy(y_hbm.at[idx_v], rows_v, sem).wait()

        lane = lax.iota(jnp.int32, _L)

        def select(j, _):
            n = ids_v[pl.ds(j * _L, _L)]
            sub = ((n >> _SH_K) & 7) * 16
            row_idx = j * _L + lane
            for o in range(O):
                vals = plsc.load_gather(rows_v, [row_idx, sub + o])
                plsc.store_scatter(sel_v, [lane * 0 + o, row_idx], vals)
            return _

        lax.fori_loop(0, _BPW // _L, select, None)
        pltpu.sync_copy(sel_v, outT_hbm.at[:, pl.ds(base, _BPW)])

    return gather_k


_gather = _make_gather()


def kernel(ids, table, W, b):
    ids = ids.astype(jnp.int32)
    b128 = jnp.tile(b, 8).reshape(1, 128)
    y = _project(table.T, W, b128)
    outT = _gather(y, ids)
    return outT.T
